# Initial kernel scaffold; baseline (speedup 1.0000x reference)
import functools

import numpy as np
import jax
import jax.numpy as jnp
from jax import lax
from jax.experimental import pallas as pl
from jax.experimental.pallas import tpu as pltpu
from jax.experimental.pallas import tpu_sc as plsc

N = 1024
D = 128
H = 64
H2 = 32
TILE = 128
NB = N // TILE
M = N * (N - 1) // 2

NUM_WORKERS = 32
CHUNK = M // NUM_WORKERS


def _prolog_body(nf_ref, wt_ref, bt_ref, w1a_ref, w1b_ref, b1_ref,
                 a_ref, b_ref):
    h = jnp.maximum(
        jnp.dot(nf_ref[...], wt_ref[...].T,
                preferred_element_type=jnp.float32) + bt_ref[...], 0.0)
    a_ref[...] = jnp.dot(h, w1a_ref[...].T,
                         preferred_element_type=jnp.float32) + b1_ref[...]
    b_ref[...] = jnp.dot(h, w1b_ref[...].T,
                         preferred_element_type=jnp.float32)


def _pair_body(a_ref, b_ref, gamma_ref, beta_ref, w2_ref, b2_ref,
               w3_ref, b3_ref, out_ref):
    bi = pl.program_id(0)
    bj = pl.program_id(1)
    a = a_ref[...]
    b = b_ref[...]
    z = a[:, None, :] + b[None, :, :]
    mu = jnp.mean(z, axis=-1, keepdims=True)
    zc = z - mu
    var = jnp.mean(zc * zc, axis=-1, keepdims=True)
    zn = zc * lax.rsqrt(var + 1e-5) * gamma_ref[...][None] \
        + beta_ref[...][None]
    zr = jnp.maximum(zn, 0.0).reshape(TILE * TILE, H)
    z2 = jnp.maximum(
        jnp.dot(zr, w2_ref[...].T, preferred_element_type=jnp.float32)
        + b2_ref[...], 0.0)
    logits = jnp.dot(z2, w3_ref[...].T,
                     preferred_element_type=jnp.float32) + b3_ref[...]
    p = jax.nn.sigmoid(logits).reshape(TILE, TILE)

    r = jnp.minimum(bi, bj)
    c = jnp.maximum(bi, bj)
    gu = r * TILE + lax.broadcasted_iota(jnp.int32, (TILE, TILE), 0)
    gv = c * TILE + lax.broadcasted_iota(jnp.int32, (TILE, TILE), 1)
    tm = jnp.where(gu < gv, p, 0.0)
    zero = jnp.zeros_like(tm)
    up = jnp.where(bi <= bj, tm, zero)
    lo = jnp.where(bi >= bj, tm.T, zero)
    out_ref[...] = up + lo


def _compute_soft_adj(node_features, Wt, bt, W1, b1, gamma, beta,
                      W2, b2, W3, b3):
    w1a = W1[:, :H]
    w1b = W1[:, H:]
    a_mat, b_mat = pl.pallas_call(
        _prolog_body,
        out_shape=(
            jax.ShapeDtypeStruct((N, H), jnp.float32),
            jax.ShapeDtypeStruct((N, H), jnp.float32),
        ),
    )(node_features, Wt, bt.reshape(1, H), w1a, w1b, b1.reshape(1, H))

    full = lambda shape: pl.BlockSpec(shape, lambda i, j: (0, 0))
    soft_adj = pl.pallas_call(
        _pair_body,
        grid=(NB, NB),
        in_specs=[
            pl.BlockSpec((TILE, H), lambda i, j: (jnp.minimum(i, j), 0)),
            pl.BlockSpec((TILE, H), lambda i, j: (jnp.maximum(i, j), 0)),
            full((1, H)),
            full((1, H)),
            full((H2, H)),
            full((1, H2)),
            full((1, H2)),
            full((1, 1)),
        ],
        out_specs=pl.BlockSpec((TILE, TILE), lambda i, j: (i, j)),
        out_shape=jax.ShapeDtypeStruct((N, N), jnp.float32),
    )(a_mat, b_mat, gamma.reshape(1, H), beta.reshape(1, H), W2,
      b2.reshape(1, H2), W3, b3.reshape(1, 1))
    return soft_adj


_SC_MESH = plsc.VectorSubcoreMesh(core_axis_name="c", subcore_axis_name="s")


@functools.partial(
    pl.kernel,
    out_type=jax.ShapeDtypeStruct((M, 1), jnp.float32),
    mesh=_SC_MESH,
    scratch_types=[
        pltpu.VMEM((CHUNK,), jnp.int32),
        pltpu.VMEM((CHUNK, 1), jnp.float32),
        pltpu.SemaphoreType.DMA,
    ],
)
def _gather_probs(padj_hbm, idx_hbm, out_hbm, idx_v, rows_v, sem):
    wid = lax.axis_index("s") * 2 + lax.axis_index("c")
    base = wid * CHUNK
    pltpu.sync_copy(idx_hbm.at[pl.ds(base, CHUNK)], idx_v)
    pltpu.async_copy(padj_hbm.at[idx_v], rows_v, sem).wait()
    pltpu.sync_copy(rows_v, out_hbm.at[pl.ds(base, CHUNK)])


def _triu_constants():
    ii, jj = np.triu_indices(N, k=1)
    flat = (ii * N + jj).astype(np.int32)
    pair = np.stack([ii, jj], axis=0)
    return jnp.asarray(flat), jnp.asarray(pair).astype(jnp.int64)


def kernel(node_features, Wt, bt, W1, b1, gamma, beta, W2, b2, W3, b3):
    soft_adj = _compute_soft_adj(node_features, Wt, bt, W1, b1, gamma,
                                 beta, W2, b2, W3, b3)
    flat_idx, pair_index = _triu_constants()
    probs = _gather_probs(soft_adj.reshape(N * N, 1), flat_idx)
    return (probs, pair_index, soft_adj)


# speedup vs baseline: 11.4292x; 11.4292x over previous
import functools

import numpy as np
import jax
import jax.numpy as jnp
from jax import lax
from jax.experimental import pallas as pl
from jax.experimental.pallas import tpu as pltpu
from jax.experimental.pallas import tpu_sc as plsc

N = 1024
D = 128
H = 64
H2 = 32
TILE = 128
NB = N // TILE
M = N * (N - 1) // 2

NUM_WORKERS = 32
CHUNK = M // NUM_WORKERS


def _prolog_body(nf_ref, wt_ref, bt_ref, w1a_ref, w1b_ref, b1_ref,
                 a_ref, b_ref):
    h = jnp.maximum(
        jnp.dot(nf_ref[...], wt_ref[...].T,
                preferred_element_type=jnp.float32) + bt_ref[...], 0.0)
    a_ref[...] = jnp.dot(h, w1a_ref[...].T,
                         preferred_element_type=jnp.float32) + b1_ref[...]
    b_ref[...] = jnp.dot(h, w1b_ref[...].T,
                         preferred_element_type=jnp.float32)


def _pair_body(a_ref, b_ref, gamma_ref, beta_ref, w2_ref, b2_ref,
               w3_ref, b3_ref, out_ref):
    bi = pl.program_id(0)
    bj = pl.program_id(1)
    a = a_ref[...]
    b = b_ref[...]
    z = a[:, None, :] + b[None, :, :]
    mu = jnp.mean(z, axis=-1, keepdims=True)
    zc = z - mu
    var = jnp.mean(zc * zc, axis=-1, keepdims=True)
    zn = zc * lax.rsqrt(var + 1e-5) * gamma_ref[...][None] \
        + beta_ref[...][None]
    zr = jnp.maximum(zn, 0.0).reshape(TILE * TILE, H)
    z2 = jnp.maximum(
        jnp.dot(zr, w2_ref[...].T, preferred_element_type=jnp.float32)
        + b2_ref[...], 0.0)
    logits = jnp.sum(z2 * w3_ref[...], axis=1, keepdims=True) + b3_ref[0, 0]
    p = jax.nn.sigmoid(logits).reshape(TILE, TILE)

    r = jnp.minimum(bi, bj)
    c = jnp.maximum(bi, bj)
    gu = r * TILE + lax.broadcasted_iota(jnp.int32, (TILE, TILE), 0)
    gv = c * TILE + lax.broadcasted_iota(jnp.int32, (TILE, TILE), 1)
    tm = jnp.where(gu < gv, p, 0.0)
    zero = jnp.zeros_like(tm)
    up = jnp.where(bi <= bj, tm, zero)
    lo = jnp.where(bi >= bj, tm.T, zero)
    out_ref[...] = up + lo


def _compute_soft_adj(node_features, Wt, bt, W1, b1, gamma, beta,
                      W2, b2, W3, b3):
    w1a = W1[:, :H]
    w1b = W1[:, H:]
    a_mat, b_mat = pl.pallas_call(
        _prolog_body,
        out_shape=(
            jax.ShapeDtypeStruct((N, H), jnp.float32),
            jax.ShapeDtypeStruct((N, H), jnp.float32),
        ),
    )(node_features, Wt, bt.reshape(1, H), w1a, w1b, b1.reshape(1, H))

    full = lambda shape: pl.BlockSpec(shape, lambda i, j: (0, 0))
    soft_adj = pl.pallas_call(
        _pair_body,
        grid=(NB, NB),
        in_specs=[
            pl.BlockSpec((TILE, H), lambda i, j: (jnp.minimum(i, j), 0)),
            pl.BlockSpec((TILE, H), lambda i, j: (jnp.maximum(i, j), 0)),
            full((1, H)),
            full((1, H)),
            full((H2, H)),
            full((1, H2)),
            full((1, H2)),
            full((1, 1)),
        ],
        out_specs=pl.BlockSpec((TILE, TILE), lambda i, j: (i, j)),
        out_shape=jax.ShapeDtypeStruct((N, N), jnp.float32),
    )(a_mat, b_mat, gamma.reshape(1, H), beta.reshape(1, H), W2,
      b2.reshape(1, H2), W3, b3.reshape(1, 1))
    return soft_adj


ROWS_PER_TILE = N // NUM_WORKERS
ROWS_WORDS = ROWS_PER_TILE * N
SEG0 = 32240
TAIL = SEG0 % 1024


@functools.cache
def _compact_probs_kernel():
    mesh = plsc.VectorSubcoreMesh(core_axis_name="c", subcore_axis_name="s")

    @functools.partial(
        pl.kernel,
        out_type=jax.ShapeDtypeStruct((M,), jnp.float32),
        mesh=mesh,
        scratch_types=[
            pltpu.VMEM((ROWS_WORDS + N,), jnp.float32),
            pltpu.VMEM((SEG0 + N,), jnp.float32),
        ],
        compiler_params=pltpu.CompilerParams(needs_layout_passes=False),
    )
    def _compact_probs(padj_hbm, out_hbm, rows_v, cbuf_v):
        t = lax.axis_index("s") * 2 + lax.axis_index("c")
        row0 = t * ROWS_PER_TILE
        seg_base = 32752 * t - 512 * t * t
        lane = lax.iota(jnp.int32, 16)

        pltpu.sync_copy(padj_hbm.at[pl.ds(row0 * N, ROWS_WORDS)],
                        rows_v.at[pl.ds(0, ROWS_WORDS)])

        def copy_row(r, _):
            src0 = r * N + row0 + r + 1
            dst0 = r * (N - 1 - row0) - r * (r - 1) // 2

            def copy_chunk(k, _):
                off = k * 16
                vals = plsc.load_gather(rows_v, [src0 + off + lane])
                plsc.store_scatter(cbuf_v, [dst0 + off + lane], vals)
                return 0

            return lax.fori_loop(0, N // 16, copy_chunk, 0)

        lax.fori_loop(0, ROWS_PER_TILE, copy_row, 0)

        def copy_out(j, _):
            pltpu.sync_copy(cbuf_v.at[pl.ds(j * 1024, 1024)],
                            out_hbm.at[pl.ds(seg_base + j * 1024, 1024)])
            return 0

        q = NUM_WORKERS - 1 - t
        lax.fori_loop(0, q, copy_out, 0)
        pltpu.sync_copy(cbuf_v.at[pl.ds(q * 1024, TAIL)],
                        out_hbm.at[pl.ds(seg_base + q * 1024, TAIL)])

    return _compact_probs


def _pair_index_constant():
    ii, jj = np.triu_indices(N, k=1)
    return jnp.asarray(np.stack([ii, jj], axis=0)).astype(jnp.int64)


def kernel(node_features, Wt, bt, W1, b1, gamma, beta, W2, b2, W3, b3):
    soft_adj = _compute_soft_adj(node_features, Wt, bt, W1, b1, gamma,
                                 beta, W2, b2, W3, b3)
    probs = _compact_probs_kernel()(soft_adj.reshape(N * N))
    return (probs.reshape(M, 1), _pair_index_constant(), soft_adj)


# speedup vs baseline: 43.2431x; 3.7836x over previous
import functools

import numpy as np
import jax
import jax.numpy as jnp
from jax import lax
from jax.experimental import pallas as pl
from jax.experimental.pallas import tpu as pltpu
from jax.experimental.pallas import tpu_sc as plsc

N = 1024
D = 128
H = 64
H2 = 32
TILE = 128
NB = N // TILE
M = N * (N - 1) // 2

NUM_WORKERS = 32
CHUNK = M // NUM_WORKERS


def _prolog_body(nf_ref, wt_ref, bt_ref, w1a_ref, w1b_ref, b1_ref,
                 gamma_ref, w2_ref, ac_ref, bct_ref, ag_ref, bgt_ref,
                 sa_ref, sbt_ref, w2b_ref):
    h = jnp.maximum(
        jnp.dot(nf_ref[...], wt_ref[...].T,
                preferred_element_type=jnp.float32) + bt_ref[...], 0.0)
    a = jnp.dot(h, w1a_ref[...].T,
                preferred_element_type=jnp.float32) + b1_ref[...]
    b = jnp.dot(h, w1b_ref[...].T, preferred_element_type=jnp.float32)
    ac = a - jnp.mean(a, axis=1, keepdims=True)
    bc = b - jnp.mean(b, axis=1, keepdims=True)
    ac_ref[...] = ac
    bct_ref[...] = bc.T
    ag_ref[...] = ac * gamma_ref[...]
    bgt_ref[...] = (bc * gamma_ref[...]).T
    sa_ref[...] = jnp.mean(ac * ac, axis=1, keepdims=True)
    sbt_ref[...] = jnp.mean(bc * bc, axis=1).reshape(1, N)
    w2b_ref[...] = jnp.broadcast_to(w2_ref[...][None], (TILE, H2, H))


def _pair_body(ac_ref, bct_ref, ag_ref, bgt_ref, sa_ref, sbt_ref, beta_ref,
               w2b_ref, b2_ref, w3_ref, b3_ref, out_ref, stash_ref):
    bi = pl.program_id(0)
    bj = pl.program_id(1)

    @pl.when(bi <= bj)
    def _compute():
        cross = jnp.dot(ac_ref[...], bct_ref[...],
                        preferred_element_type=jnp.float32)
        var = sa_ref[...] + sbt_ref[...] + (2.0 / H) * cross
        s = lax.rsqrt(var + 1e-5)
        q = ag_ref[...][:, :, None] + bgt_ref[...][None, :, :]
        zn = q * s[:, None, :] + beta_ref[...][None, :, :]
        zr = jnp.maximum(zn, 0.0)
        z2 = lax.dot_general(w2b_ref[...], zr, (((2,), (1,)), ((0,), (0,))),
                             preferred_element_type=jnp.float32)
        z2 = jnp.maximum(z2 + b2_ref[...][None, :, :], 0.0)
        logits = jnp.sum(z2 * w3_ref[...][None, :, :], axis=1) \
            + b3_ref[0, 0]
        p = jax.nn.sigmoid(logits)
        gu = bi * TILE + lax.broadcasted_iota(jnp.int32, (TILE, TILE), 0)
        gv = bj * TILE + lax.broadcasted_iota(jnp.int32, (TILE, TILE), 1)
        tm = jnp.where(gu < gv, p, 0.0)
        slot = bi * NB - bi * (bi - 1) // 2 + (bj - bi)
        stash_ref[slot] = tm
        out_ref[...] = jnp.where(bi == bj, tm + tm.T, tm)

    @pl.when(bi > bj)
    def _mirror():
        slot = bj * NB - bj * (bj - 1) // 2 + (bi - bj)
        out_ref[...] = stash_ref[slot].T


def _compute_soft_adj(node_features, Wt, bt, W1, b1, gamma, beta,
                      W2, b2, W3, b3):
    w1a = W1[:, :H]
    w1b = W1[:, H:]
    nh = jax.ShapeDtypeStruct((N, H), jnp.float32)
    hn = jax.ShapeDtypeStruct((H, N), jnp.float32)
    ac, bct, ag, bgt, sa, sbt, w2b = pl.pallas_call(
        _prolog_body,
        out_shape=(nh, hn, nh, hn,
                   jax.ShapeDtypeStruct((N, 1), jnp.float32),
                   jax.ShapeDtypeStruct((1, N), jnp.float32),
                   jax.ShapeDtypeStruct((TILE, H2, H), jnp.float32)),
    )(node_features, Wt, bt.reshape(1, H), w1a, w1b, b1.reshape(1, H),
      gamma.reshape(1, H), W2)

    rmap = lambda i, j: (jnp.minimum(i, j), 0)
    cmap = lambda i, j: (0, jnp.maximum(i, j))
    full = lambda shape: pl.BlockSpec(shape, lambda i, j: (0, 0))
    soft_adj = pl.pallas_call(
        _pair_body,
        grid=(NB, NB),
        in_specs=[
            pl.BlockSpec((TILE, H), rmap),
            pl.BlockSpec((H, TILE), cmap),
            pl.BlockSpec((TILE, H), rmap),
            pl.BlockSpec((H, TILE), cmap),
            pl.BlockSpec((TILE, 1), rmap),
            pl.BlockSpec((1, TILE), cmap),
            full((H, 1)),
            pl.BlockSpec((TILE, H2, H), lambda i, j: (0, 0, 0)),
            full((H2, 1)),
            full((H2, 1)),
            full((1, 1)),
        ],
        out_specs=pl.BlockSpec((TILE, TILE), lambda i, j: (i, j)),
        out_shape=jax.ShapeDtypeStruct((N, N), jnp.float32),
        scratch_shapes=[
            pltpu.VMEM((NB * (NB + 1) // 2, TILE, TILE), jnp.float32)
        ],
    )(ac, bct, ag, bgt, sa, sbt, beta.reshape(H, 1), w2b,
      b2.reshape(H2, 1), W3.reshape(H2, 1), b3.reshape(1, 1))
    return soft_adj


ROWS_PER_TILE = N // NUM_WORKERS
ROWS_WORDS = ROWS_PER_TILE * N
SEG0 = 32240
TAIL = SEG0 % 1024


@functools.cache
def _compact_probs_kernel():
    mesh = plsc.VectorSubcoreMesh(core_axis_name="c", subcore_axis_name="s")

    @functools.partial(
        pl.kernel,
        out_type=jax.ShapeDtypeStruct((M,), jnp.float32),
        mesh=mesh,
        scratch_types=[
            pltpu.VMEM((ROWS_WORDS + N,), jnp.float32),
            pltpu.VMEM((SEG0 + N,), jnp.float32),
        ],
        compiler_params=pltpu.CompilerParams(needs_layout_passes=False),
    )
    def _compact_probs(padj_hbm, out_hbm, rows_v, cbuf_v):
        t = lax.axis_index("s") * 2 + lax.axis_index("c")
        row0 = t * ROWS_PER_TILE
        seg_base = 32752 * t - 512 * t * t
        lane = lax.iota(jnp.int32, 16)

        pltpu.sync_copy(padj_hbm.at[pl.ds(row0 * N, ROWS_WORDS)],
                        rows_v.at[pl.ds(0, ROWS_WORDS)])

        def copy_row(r, _):
            src0 = r * N + row0 + r + 1
            dst0 = r * (N - 1 - row0) - r * (r - 1) // 2

            def copy_chunk(k, _):
                off = k * 16
                vals = plsc.load_gather(rows_v, [src0 + off + lane])
                plsc.store_scatter(cbuf_v, [dst0 + off + lane], vals)
                return 0

            return lax.fori_loop(0, N // 16, copy_chunk, 0)

        lax.fori_loop(0, ROWS_PER_TILE, copy_row, 0)

        def copy_out(j, _):
            pltpu.sync_copy(cbuf_v.at[pl.ds(j * 1024, 1024)],
                            out_hbm.at[pl.ds(seg_base + j * 1024, 1024)])
            return 0

        q = NUM_WORKERS - 1 - t
        lax.fori_loop(0, q, copy_out, 0)
        pltpu.sync_copy(cbuf_v.at[pl.ds(q * 1024, TAIL)],
                        out_hbm.at[pl.ds(seg_base + q * 1024, TAIL)])

    return _compact_probs


def _pair_index_constant():
    ii, jj = np.triu_indices(N, k=1)
    return jnp.asarray(np.stack([ii, jj], axis=0)).astype(jnp.int64)


def kernel(node_features, Wt, bt, W1, b1, gamma, beta, W2, b2, W3, b3):
    soft_adj = _compute_soft_adj(node_features, Wt, bt, W1, b1, gamma,
                                 beta, W2, b2, W3, b3)
    probs = _compact_probs_kernel()(soft_adj.reshape(N * N))
    return (probs.reshape(M, 1), _pair_index_constant(), soft_adj)
